# baseline (speedup 1.0000x reference)
import functools

import jax
import jax.numpy as jnp
from jax import lax
from jax.experimental import pallas as pl
from jax.experimental.pallas import tpu as pltpu
from jax.experimental.pallas import tpu_sc as plsc

N_NODES = 10000
D = 128
NC = 2
NS = 16
NW = NC * NS
K = 128
N_EDGES = 320000
CHUNKS = 80
NEP = NW * CHUNKS * K
N_PAD = 10240
ROWS_PER_TILE = N_PAD // NS

_MESH = plsc.VectorSubcoreMesh(core_axis_name="c", subcore_axis_name="s")


@functools.partial(
    pl.kernel,
    out_type=jax.ShapeDtypeStruct((NC, N_PAD), jnp.float32),
    mesh=_MESH,
    scratch_types=[
        pltpu.VMEM((CHUNKS, K), jnp.int32),
        pltpu.VMEM((K,), jnp.float32),
        pltpu.VMEM((ROWS_PER_TILE,), jnp.float32),
        pltpu.VMEM_SHARED((N_PAD,), jnp.float32),
    ],
)
def _sc_degree(dst_hbm, out_hbm, dst_v, ones_v, z_v, acc):
    c = lax.axis_index("c")
    s = lax.axis_index("s")
    wid = s * NC + c
    one16 = jnp.full((16,), 1.0, jnp.float32)
    zero16 = jnp.zeros((16,), jnp.float32)
    for k in range(K // 16):
        ones_v[pl.ds(16 * k, 16)] = one16

    @pl.loop(0, ROWS_PER_TILE // 16)
    def _(i):
        z_v[pl.ds(i * 16, 16)] = zero16

    pltpu.sync_copy(z_v, acc.at[pl.ds(s * ROWS_PER_TILE, ROWS_PER_TILE)])
    plsc.subcore_barrier()
    pltpu.sync_copy(dst_hbm.at[wid], dst_v)

    @pl.loop(0, CHUNKS)
    def _(j):
        pltpu.sync_copy(ones_v, acc.at[dst_v.at[j]], add=True)

    plsc.subcore_barrier()
    pltpu.sync_copy(acc.at[pl.ds(s * ROWS_PER_TILE, ROWS_PER_TILE)],
                    out_hbm.at[c, pl.ds(s * ROWS_PER_TILE, ROWS_PER_TILE)])


CH_A = 80
CH_B = 80
SEG = 40


@functools.partial(
    pl.kernel,
    out_type=jax.ShapeDtypeStruct((NC, N_PAD, D), jnp.float32),
    mesh=_MESH,
    scratch_types=[
        pltpu.VMEM((SEG, K), jnp.int32),
        pltpu.VMEM((SEG, K), jnp.int32),
        pltpu.VMEM((K, D), jnp.float32),
        pltpu.VMEM((K, D), jnp.float32),
        pltpu.SemaphoreType.DMA,
        pltpu.SemaphoreType.DMA,
        pltpu.SemaphoreType.DMA,
        pltpu.SemaphoreType.DMA,
        pltpu.VMEM_SHARED((N_PAD, D), jnp.float32),
    ],
)
def _sc_aggregate(h_hbm, srca_hbm, dsta_hbm, srcb_hbm, dstb_hbm, out_hbm,
                  src_v, dst_v, rows0, rows1, sem0, sem1, ssem0, ssem1, acc):
    c = lax.axis_index("c")
    s = lax.axis_index("s")
    zero16 = jnp.zeros((16,), jnp.float32)
    bufs = (rows0, rows1)
    sems = (sem0, sem1)
    ssems = (ssem0, ssem1)

    with jax.named_scope("agg_zero"):
        @pl.loop(0, K)
        def _(r):
            for k in range(D // 16):
                rows0[r, pl.ds(16 * k, 16)] = zero16

        for b in range(ROWS_PER_TILE // K):
            pltpu.sync_copy(rows0, acc.at[pl.ds(s * ROWS_PER_TILE + b * K, K)])
    plsc.subcore_barrier()

    def run_segments(src_hbm, dst_hbm, nseg):
        for seg in range(nseg):
            pltpu.sync_copy(src_hbm.at[s, pl.ds(seg * SEG, SEG)], src_v)
            pltpu.sync_copy(dst_hbm.at[s, pl.ds(seg * SEG, SEG)], dst_v)
            pltpu.async_copy(h_hbm.at[src_v.at[0]], rows0, sem0)
            pltpu.async_copy(h_hbm.at[src_v.at[1]], rows1, sem1)

            @pl.loop(0, SEG, step=2)
            def _(j):
                for b in range(2):
                    jj = j + b
                    pltpu.make_async_copy(
                        h_hbm.at[src_v.at[jj]], bufs[b], sems[b]).wait()
                    pltpu.async_copy(
                        bufs[b], acc.at[dst_v.at[jj]], ssems[b], add=True)
                for b in range(2):
                    jj = j + b
                    pltpu.make_async_copy(
                        bufs[b], acc.at[dst_v.at[jj]], ssems[b]).wait()

                    @pl.when(jj + 2 < SEG)
                    def _():
                        pltpu.async_copy(
                            h_hbm.at[src_v.at[jj + 2]], bufs[b], sems[b])

    with jax.named_scope("agg_ring"):
        @pl.when(c == 0)
        def _():
            run_segments(srca_hbm, dsta_hbm, CH_A // SEG)

        @pl.when(c == 1)
        def _():
            run_segments(srcb_hbm, dstb_hbm, CH_B // SEG)

    plsc.subcore_barrier()
    with jax.named_scope("agg_out"):
        for b in range(ROWS_PER_TILE // K):
            pltpu.sync_copy(acc.at[pl.ds(s * ROWS_PER_TILE + b * K, K)], rows0)
            pltpu.sync_copy(rows0,
                            out_hbm.at[c, pl.ds(s * ROWS_PER_TILE + b * K, K)])


R_BLK = 1000


def _tc_dense_body(relu, p0, p1, pd0, pd1, h, wl, bl, wr, out):
    dinv = 1.0 / jnp.maximum(pd0[...] + pd1[...], 1.0)
    agg = (p0[...] + p1[...]) * dinv
    o = (jnp.dot(agg, wl[...], preferred_element_type=jnp.float32) + bl[...]
         + jnp.dot(h[...], wr[...], preferred_element_type=jnp.float32))
    n = jnp.sqrt(jnp.sum(o * o, axis=-1, keepdims=True))
    o = o / jnp.maximum(n, 1e-12)
    if relu:
        o = jnp.maximum(o, 0.0)
    out[...] = o


def _make_tc_dense(relu):
    row_spec = pl.BlockSpec((R_BLK, D), lambda i: (i, 0))
    col_spec = pl.BlockSpec((R_BLK, 1), lambda i: (i, 0))
    w_spec = pl.BlockSpec((D, D), lambda i: (0, 0))
    b_spec = pl.BlockSpec((1, D), lambda i: (0, 0))
    return pl.pallas_call(
        functools.partial(_tc_dense_body, relu),
        grid=(N_NODES // R_BLK,),
        in_specs=[row_spec, row_spec, col_spec, col_spec, row_spec,
                  w_spec, b_spec, w_spec],
        out_specs=row_spec,
        out_shape=jax.ShapeDtypeStruct((N_NODES, D), jnp.float32),
    )


_TC_DENSE = {True: _make_tc_dense(True), False: _make_tc_dense(False)}


def kernel(x, edge_index, Wl0, bl0, Wr0, Wl1, bl1, Wr1, Wl2, bl2, Wr2):
    ei = edge_index.astype(jnp.int32)
    pad = NEP - N_EDGES
    src_fill = jnp.arange(pad, dtype=jnp.int32) % N_NODES
    src_p = jnp.concatenate([ei[0], src_fill])
    trash = N_NODES + (jnp.arange(pad, dtype=jnp.int32) % (N_PAD - N_NODES))
    dst_p = jnp.concatenate([ei[1], trash])
    dst3 = dst_p.reshape(NW, CHUNKS, K)
    n_a = NS * CH_A * K
    srca = src_p[:n_a].reshape(NS, CH_A, K)
    dsta = dst_p[:n_a].reshape(NS, CH_A, K)
    srcb = src_p[n_a:].reshape(NS, CH_B, K)
    dstb = dst_p[n_a:].reshape(NS, CH_B, K)

    pdeg = _sc_degree(dst3)
    pd0 = pdeg[0, :N_NODES][:, None]
    pd1 = pdeg[1, :N_NODES][:, None]

    h = x
    for Wl, bl, Wr, relu in ((Wl0, bl0, Wr0, True),
                             (Wl1, bl1, Wr1, True),
                             (Wl2, bl2, Wr2, False)):
        p = _sc_aggregate(h, srca, dsta, srcb, dstb)
        h = _TC_DENSE[relu](p[0, :N_NODES], p[1, :N_NODES], pd0, pd1, h,
                            Wl, bl.reshape(1, D), Wr)
    return h


# speedup vs baseline: 1.2329x; 1.2329x over previous
import functools

import jax
import jax.numpy as jnp
from jax import lax
from jax.experimental import pallas as pl
from jax.experimental.pallas import tpu as pltpu
from jax.experimental.pallas import tpu_sc as plsc

N_NODES = 10000
D = 128
NC = 2
NS = 16
NW = NC * NS
K = 128
N_EDGES = 320000
CHUNKS = 80
NEP = NW * CHUNKS * K
N_PAD = 10240
ROWS_PER_TILE = N_PAD // NS

_MESH = plsc.VectorSubcoreMesh(core_axis_name="c", subcore_axis_name="s")


CH_A = 80
CH_B = 80
SEG = 40


def _make_sc_aggregate(with_deg):
    out_type = [jax.ShapeDtypeStruct((NC, N_PAD, D), jnp.float32)]
    scratch = [
        pltpu.VMEM((SEG, K), jnp.int32),
        pltpu.VMEM((SEG, K), jnp.int32),
        pltpu.VMEM((K, D), jnp.float32),
        pltpu.VMEM((K, D), jnp.float32),
        pltpu.SemaphoreType.DMA,
        pltpu.SemaphoreType.DMA,
        pltpu.VMEM_SHARED((N_PAD, D), jnp.float32),
    ]
    if with_deg:
        out_type.append(jax.ShapeDtypeStruct((NC, N_PAD), jnp.float32))
        scratch.append(pltpu.VMEM((K,), jnp.float32))
        scratch.append(pltpu.VMEM_SHARED((N_PAD,), jnp.float32))

    def body(h_hbm, srca_hbm, dsta_hbm, srcb_hbm, dstb_hbm, out_hbm, *rest):
        if with_deg:
            (dout_hbm, src_v, dst_v, rows0, rows1, sem0, sem1, acc,
             ones_v, dacc) = rest
        else:
            src_v, dst_v, rows0, rows1, sem0, sem1, acc = rest
        c = lax.axis_index("c")
        s = lax.axis_index("s")
        zero16 = jnp.zeros((16,), jnp.float32)
        bufs = (rows0, rows1)
        sems = (sem0, sem1)

        with jax.named_scope("agg_zero"):
            @pl.loop(0, K)
            def _(r):
                for k in range(D // 16):
                    rows0[r, pl.ds(16 * k, 16)] = zero16

            for b in range(ROWS_PER_TILE // K):
                pltpu.sync_copy(rows0,
                                acc.at[pl.ds(s * ROWS_PER_TILE + b * K, K)])
            if with_deg:
                one16 = jnp.full((16,), 1.0, jnp.float32)
                for k in range(K // 16):
                    ones_v[pl.ds(16 * k, 16)] = one16
                for b in range(ROWS_PER_TILE // K):
                    pltpu.sync_copy(
                        rows0.at[0],
                        dacc.at[pl.ds(s * ROWS_PER_TILE + b * K, K)])
        plsc.subcore_barrier()

        def run_segments(src_hbm, dst_hbm, nseg):
            for seg in range(nseg):
                pltpu.sync_copy(src_hbm.at[s, pl.ds(seg * SEG, SEG)], src_v)
                pltpu.sync_copy(dst_hbm.at[s, pl.ds(seg * SEG, SEG)], dst_v)
                pltpu.async_copy(h_hbm.at[src_v.at[0]], rows0, sem0)
                pltpu.async_copy(h_hbm.at[src_v.at[1]], rows1, sem1)

                @pl.loop(0, SEG, step=2)
                def _(j):
                    for b in range(2):
                        jj = j + b
                        pltpu.make_async_copy(
                            h_hbm.at[src_v.at[jj]], bufs[b], sems[b]).wait()
                        pltpu.sync_copy(bufs[b], acc.at[dst_v.at[jj]],
                                        add=True)

                        @pl.when(jj + 2 < SEG)
                        def _():
                            pltpu.async_copy(
                                h_hbm.at[src_v.at[jj + 2]], bufs[b], sems[b])

                if with_deg:
                    @pl.loop(0, SEG)
                    def _(j):
                        pltpu.sync_copy(ones_v, dacc.at[dst_v.at[j]],
                                        add=True)

        with jax.named_scope("agg_ring"):
            @pl.when(c == 0)
            def _():
                run_segments(srca_hbm, dsta_hbm, CH_A // SEG)

            @pl.when(c == 1)
            def _():
                run_segments(srcb_hbm, dstb_hbm, CH_B // SEG)

        plsc.subcore_barrier()
        with jax.named_scope("agg_out"):
            for b in range(ROWS_PER_TILE // K):
                pltpu.sync_copy(acc.at[pl.ds(s * ROWS_PER_TILE + b * K, K)],
                                rows0)
                pltpu.sync_copy(
                    rows0, out_hbm.at[c, pl.ds(s * ROWS_PER_TILE + b * K, K)])
            if with_deg:
                pltpu.sync_copy(
                    dacc.at[pl.ds(s * ROWS_PER_TILE, ROWS_PER_TILE)],
                    dout_hbm.at[c, pl.ds(s * ROWS_PER_TILE, ROWS_PER_TILE)])

    return pl.kernel(body, out_type=out_type, mesh=_MESH,
                     scratch_types=scratch)


_SC_AGG = _make_sc_aggregate(False)
_SC_AGG_DEG = _make_sc_aggregate(True)


R_BLK = 1000


def _tc_dense_body(relu, p0, p1, pd0, pd1, h, wl, bl, wr, out):
    dinv = 1.0 / jnp.maximum(pd0[...] + pd1[...], 1.0)
    agg = (p0[...] + p1[...]) * dinv
    o = (jnp.dot(agg, wl[...], preferred_element_type=jnp.float32) + bl[...]
         + jnp.dot(h[...], wr[...], preferred_element_type=jnp.float32))
    n = jnp.sqrt(jnp.sum(o * o, axis=-1, keepdims=True))
    o = o / jnp.maximum(n, 1e-12)
    if relu:
        o = jnp.maximum(o, 0.0)
    out[...] = o


def _make_tc_dense(relu):
    row_spec = pl.BlockSpec((R_BLK, D), lambda i: (i, 0))
    col_spec = pl.BlockSpec((R_BLK, 1), lambda i: (i, 0))
    w_spec = pl.BlockSpec((D, D), lambda i: (0, 0))
    b_spec = pl.BlockSpec((1, D), lambda i: (0, 0))
    return pl.pallas_call(
        functools.partial(_tc_dense_body, relu),
        grid=(N_NODES // R_BLK,),
        in_specs=[row_spec, row_spec, col_spec, col_spec, row_spec,
                  w_spec, b_spec, w_spec],
        out_specs=row_spec,
        out_shape=jax.ShapeDtypeStruct((N_NODES, D), jnp.float32),
    )


_TC_DENSE = {True: _make_tc_dense(True), False: _make_tc_dense(False)}


def kernel(x, edge_index, Wl0, bl0, Wr0, Wl1, bl1, Wr1, Wl2, bl2, Wr2):
    ei = edge_index.astype(jnp.int32)
    pad = NEP - N_EDGES
    src_fill = jnp.arange(pad, dtype=jnp.int32) % N_NODES
    src_p = jnp.concatenate([ei[0], src_fill])
    trash = N_NODES + (jnp.arange(pad, dtype=jnp.int32) % (N_PAD - N_NODES))
    dst_p = jnp.concatenate([ei[1], trash])
    n_a = NS * CH_A * K
    srca = src_p[:n_a].reshape(NS, CH_A, K)
    dsta = dst_p[:n_a].reshape(NS, CH_A, K)
    srcb = src_p[n_a:].reshape(NS, CH_B, K)
    dstb = dst_p[n_a:].reshape(NS, CH_B, K)

    h = x
    pd0 = pd1 = None
    for i, (Wl, bl, Wr, relu) in enumerate(((Wl0, bl0, Wr0, True),
                                            (Wl1, bl1, Wr1, True),
                                            (Wl2, bl2, Wr2, False))):
        if i == 0:
            p, pdeg = _SC_AGG_DEG(h, srca, dsta, srcb, dstb)
            pd0 = pdeg[0, :N_NODES][:, None]
            pd1 = pdeg[1, :N_NODES][:, None]
        else:
            (p,) = _SC_AGG(h, srca, dsta, srcb, dstb)
        h = _TC_DENSE[relu](p[0, :N_NODES], p[1, :N_NODES], pd0, pd1, h,
                            Wl, bl.reshape(1, D), Wr)
    return h


# speedup vs baseline: 1.2486x; 1.0127x over previous
import functools

import jax
import jax.numpy as jnp
from jax import lax
from jax.experimental import pallas as pl
from jax.experimental.pallas import tpu as pltpu
from jax.experimental.pallas import tpu_sc as plsc

N_NODES = 10000
D = 128
NC = 2
NS = 16
NW = NC * NS
K = 128
N_EDGES = 320000
CHUNKS = 80
NEP = NW * CHUNKS * K
N_PAD = 10240
ROWS_PER_TILE = N_PAD // NS

_MESH = plsc.VectorSubcoreMesh(core_axis_name="c", subcore_axis_name="s")


@functools.partial(
    pl.kernel,
    out_type=jax.ShapeDtypeStruct((NC, N_PAD), jnp.float32),
    mesh=_MESH,
    scratch_types=[
        pltpu.VMEM((CHUNKS, K), jnp.int32),
        pltpu.VMEM((K,), jnp.float32),
        pltpu.VMEM((ROWS_PER_TILE,), jnp.float32),
        pltpu.VMEM_SHARED((N_PAD,), jnp.float32),
    ],
)
def _sc_degree(dst_hbm, out_hbm, dst_v, ones_v, z_v, acc):
    c = lax.axis_index("c")
    s = lax.axis_index("s")
    wid = s * NC + c
    one16 = jnp.full((16,), 1.0, jnp.float32)
    zero16 = jnp.zeros((16,), jnp.float32)
    for k in range(K // 16):
        ones_v[pl.ds(16 * k, 16)] = one16

    @pl.loop(0, ROWS_PER_TILE // 16)
    def _(i):
        z_v[pl.ds(i * 16, 16)] = zero16

    pltpu.sync_copy(z_v, acc.at[pl.ds(s * ROWS_PER_TILE, ROWS_PER_TILE)])
    plsc.subcore_barrier()
    pltpu.sync_copy(dst_hbm.at[wid], dst_v)

    @pl.loop(0, CHUNKS)
    def _(j):
        pltpu.sync_copy(ones_v, acc.at[dst_v.at[j]], add=True)

    plsc.subcore_barrier()
    pltpu.sync_copy(acc.at[pl.ds(s * ROWS_PER_TILE, ROWS_PER_TILE)],
                    out_hbm.at[c, pl.ds(s * ROWS_PER_TILE, ROWS_PER_TILE)])


CH_A = 80
CH_B = 80
SEG = 40


@functools.partial(
    pl.kernel,
    out_type=jax.ShapeDtypeStruct((NC, N_PAD, D), jnp.float32),
    mesh=_MESH,
    scratch_types=[
        pltpu.VMEM((SEG, K), jnp.int32),
        pltpu.VMEM((SEG, K), jnp.int32),
        pltpu.VMEM((K, D), jnp.float32),
        pltpu.VMEM((K, D), jnp.float32),
        pltpu.SemaphoreType.DMA,
        pltpu.SemaphoreType.DMA,
        pltpu.VMEM_SHARED((N_PAD, D), jnp.float32),
    ],
)
def _sc_aggregate(h_hbm, srca_hbm, dsta_hbm, srcb_hbm, dstb_hbm, out_hbm,
                  src_v, dst_v, rows0, rows1, sem0, sem1, acc):
    c = lax.axis_index("c")
    s = lax.axis_index("s")
    zero16 = jnp.zeros((16,), jnp.float32)
    bufs = (rows0, rows1)
    sems = (sem0, sem1)

    with jax.named_scope("agg_zero"):
        @pl.loop(0, K)
        def _(r):
            for k in range(D // 16):
                rows0[r, pl.ds(16 * k, 16)] = zero16

        for b in range(ROWS_PER_TILE // K):
            pltpu.sync_copy(rows0, acc.at[pl.ds(s * ROWS_PER_TILE + b * K, K)])
    plsc.subcore_barrier()

    def run_segments(src_hbm, dst_hbm, nseg):
        for seg in range(nseg):
            pltpu.sync_copy(src_hbm.at[s, pl.ds(seg * SEG, SEG)], src_v)
            pltpu.sync_copy(dst_hbm.at[s, pl.ds(seg * SEG, SEG)], dst_v)
            pltpu.async_copy(h_hbm.at[src_v.at[0]], rows0, sem0)
            pltpu.async_copy(h_hbm.at[src_v.at[1]], rows1, sem1)

            @pl.loop(0, SEG, step=2)
            def _(j):
                for b in range(2):
                    jj = j + b
                    pltpu.make_async_copy(
                        h_hbm.at[src_v.at[jj]], bufs[b], sems[b]).wait()
                    pltpu.sync_copy(bufs[b], acc.at[dst_v.at[jj]], add=True)

                    @pl.when(jj + 2 < SEG)
                    def _():
                        pltpu.async_copy(
                            h_hbm.at[src_v.at[jj + 2]], bufs[b], sems[b])

    with jax.named_scope("agg_ring"):
        @pl.when(c == 0)
        def _():
            run_segments(srca_hbm, dsta_hbm, CH_A // SEG)

        @pl.when(c == 1)
        def _():
            run_segments(srcb_hbm, dstb_hbm, CH_B // SEG)

    plsc.subcore_barrier()
    with jax.named_scope("agg_out"):
        pltpu.sync_copy(acc.at[pl.ds(s * ROWS_PER_TILE, ROWS_PER_TILE)],
                        out_hbm.at[c, pl.ds(s * ROWS_PER_TILE, ROWS_PER_TILE)])


R_BLK = 1000


def _tc_dense_body(relu, p0, p1, pd0, pd1, h, wl, bl, wr, out):
    dinv = 1.0 / jnp.maximum(pd0[...] + pd1[...], 1.0)
    agg = (p0[...] + p1[...]) * dinv
    o = (jnp.dot(agg, wl[...], preferred_element_type=jnp.float32) + bl[...]
         + jnp.dot(h[...], wr[...], preferred_element_type=jnp.float32))
    n = jnp.sqrt(jnp.sum(o * o, axis=-1, keepdims=True))
    o = o / jnp.maximum(n, 1e-12)
    if relu:
        o = jnp.maximum(o, 0.0)
    out[...] = o


def _make_tc_dense(relu):
    row_spec = pl.BlockSpec((R_BLK, D), lambda i: (i, 0))
    col_spec = pl.BlockSpec((R_BLK, 1), lambda i: (i, 0))
    w_spec = pl.BlockSpec((D, D), lambda i: (0, 0))
    b_spec = pl.BlockSpec((1, D), lambda i: (0, 0))
    return pl.pallas_call(
        functools.partial(_tc_dense_body, relu),
        grid=(N_NODES // R_BLK,),
        in_specs=[row_spec, row_spec, col_spec, col_spec, row_spec,
                  w_spec, b_spec, w_spec],
        out_specs=row_spec,
        out_shape=jax.ShapeDtypeStruct((N_NODES, D), jnp.float32),
    )


_TC_DENSE = {True: _make_tc_dense(True), False: _make_tc_dense(False)}


def kernel(x, edge_index, Wl0, bl0, Wr0, Wl1, bl1, Wr1, Wl2, bl2, Wr2):
    ei = edge_index.astype(jnp.int32)
    pad = NEP - N_EDGES
    src_fill = jnp.arange(pad, dtype=jnp.int32) % N_NODES
    src_p = jnp.concatenate([ei[0], src_fill])
    trash = N_NODES + (jnp.arange(pad, dtype=jnp.int32) % (N_PAD - N_NODES))
    dst_p = jnp.concatenate([ei[1], trash])
    dst3 = dst_p.reshape(NW, CHUNKS, K)
    n_a = NS * CH_A * K
    srca = src_p[:n_a].reshape(NS, CH_A, K)
    dsta = dst_p[:n_a].reshape(NS, CH_A, K)
    srcb = src_p[n_a:].reshape(NS, CH_B, K)
    dstb = dst_p[n_a:].reshape(NS, CH_B, K)

    pdeg = _sc_degree(dst3)
    pd0 = pdeg[0, :N_NODES][:, None]
    pd1 = pdeg[1, :N_NODES][:, None]

    h = x
    for Wl, bl, Wr, relu in ((Wl0, bl0, Wr0, True),
                             (Wl1, bl1, Wr1, True),
                             (Wl2, bl2, Wr2, False)):
        p = _sc_aggregate(h, srca, dsta, srcb, dstb)
        h = _TC_DENSE[relu](p[0, :N_NODES], p[1, :N_NODES], pd0, pd1, h,
                            Wl, bl.reshape(1, D), Wr)
    return h
